# baseline (speedup 1.0000x reference)
import functools

import jax
import jax.numpy as jnp
from jax import lax
from jax.experimental import pallas as pl
from jax.experimental.pallas import tpu as pltpu
from jax.experimental.pallas import tpu_sc as plsc

N_TOK = 2048
DIM = 768
NEXP = 8
HID = 4 * DIM
TOK_TILE = 256
VOCAB_TILE = 2048
PROJ_TOK = 512
PROJ_VOC = 4096


def _sc_gather(table, idx):
    info = plsc.get_sparse_core_info()
    nw = info.num_cores * info.num_subcores
    n = idx.shape[0]
    d = table.shape[1]
    b_per_w = n // nw
    mesh = plsc.VectorSubcoreMesh(core_axis_name="c", subcore_axis_name="s")

    @functools.partial(
        pl.kernel,
        mesh=mesh,
        out_type=jax.ShapeDtypeStruct((n, d), jnp.float32),
        scratch_types=[
            pltpu.VMEM((b_per_w,), jnp.int32),
            pltpu.VMEM((b_per_w, d), jnp.float32),
            pltpu.SemaphoreType.DMA,
        ],
    )
    def k(table_hbm, idx_hbm, out_hbm, idx_v, rows_v, sem):
        wid = lax.axis_index("s") * info.num_cores + lax.axis_index("c")
        base = wid * b_per_w
        pltpu.sync_copy(idx_hbm.at[pl.ds(base, b_per_w)], idx_v)
        pltpu.async_copy(table_hbm.at[idx_v], rows_v, sem).wait()
        pltpu.sync_copy(rows_v, out_hbm.at[pl.ds(base, b_per_w)])

    return k(table, idx)


def _router_body(h_ref, wg_ref, cmb_ref, aux_ref):
    h = h_ref[...]
    wg = wg_ref[...]
    logits = jnp.dot(h, wg, preferred_element_type=jnp.float32)
    probs = jax.nn.softmax(logits, axis=-1)
    gate = jnp.max(probs, axis=-1, keepdims=True)
    ids = lax.broadcasted_iota(jnp.int32, (N_TOK, NEXP), 1)
    eidx = jnp.min(jnp.where(probs >= gate, ids, NEXP), axis=-1, keepdims=True)
    oh = (ids == eidx).astype(jnp.float32)
    f = jnp.mean(oh, axis=0, keepdims=True)
    p_mean = jnp.mean(probs, axis=0, keepdims=True)
    aux_ref[...] = NEXP * jnp.sum(f * p_mean, axis=1, keepdims=True)
    cmb_ref[...] = oh * gate


def _router(h, wg):
    return pl.pallas_call(
        _router_body,
        out_shape=(
            jax.ShapeDtypeStruct((N_TOK, NEXP), jnp.float32),
            jax.ShapeDtypeStruct((1, 1), jnp.float32),
        ),
    )(h, wg)


def _moe_body(h_ref, cmb_ref, w1_ref, b1_ref, w2_ref, b2_ref, y_ref, acc_ref):
    e = pl.program_id(0)
    t = pl.program_id(1)
    hb = h_ref[...].astype(jnp.bfloat16)
    h1 = jnp.dot(hb, w1_ref[0].astype(jnp.bfloat16),
                 preferred_element_type=jnp.float32) + b1_ref[0]
    a = jax.nn.gelu(h1).astype(jnp.bfloat16)
    eo = jnp.dot(a, w2_ref[0].astype(jnp.bfloat16),
                 preferred_element_type=jnp.float32) + b2_ref[0]
    lane = lax.broadcasted_iota(jnp.int32, (TOK_TILE, NEXP), 1)
    cmb_e = jnp.sum(jnp.where(lane == e, cmb_ref[...], 0.0), axis=1,
                    keepdims=True)
    contrib = eo * cmb_e
    sl = pl.ds(t * TOK_TILE, TOK_TILE)

    @pl.when(e == 0)
    def _():
        acc_ref[sl, :] = contrib

    @pl.when(e > 0)
    def _():
        acc_ref[sl, :] = acc_ref[sl, :] + contrib

    @pl.when(e == NEXP - 1)
    def _():
        y_ref[...] = acc_ref[sl, :].astype(jnp.bfloat16)


def _moe(h, cmb, w1, b1, w2, b2):
    nt = N_TOK // TOK_TILE
    return pl.pallas_call(
        _moe_body,
        grid=(NEXP, nt),
        in_specs=[
            pl.BlockSpec((TOK_TILE, DIM), lambda e, t: (t, 0)),
            pl.BlockSpec((TOK_TILE, NEXP), lambda e, t: (t, 0)),
            pl.BlockSpec((1, DIM, HID), lambda e, t: (e, 0, 0)),
            pl.BlockSpec((1, 1, HID), lambda e, t: (e, 0, 0)),
            pl.BlockSpec((1, HID, DIM), lambda e, t: (e, 0, 0)),
            pl.BlockSpec((1, 1, DIM), lambda e, t: (e, 0, 0)),
        ],
        out_specs=pl.BlockSpec((TOK_TILE, DIM), lambda e, t: (t, 0)),
        out_shape=jax.ShapeDtypeStruct((N_TOK, DIM), jnp.bfloat16),
        scratch_shapes=[pltpu.VMEM((N_TOK, DIM), jnp.float32)],
    )(h, cmb, w1.reshape(NEXP, DIM, HID), b1.reshape(NEXP, 1, HID),
      w2.reshape(NEXP, HID, DIM), b2.reshape(NEXP, 1, DIM))


PV = 8192
PT = 64
NSL = N_TOK // PT
NFULL = 12
REM = 1696
ORING = 2


KQ = DIM // 4


def _proj_manual_body(y_ref, wo_hbm, bo_ref, out_hbm,
                      outr_ref, worh_ref, wobf_ref, worem_ref, outrem_ref,
                      out_sem, wo_sem, rem_sem):

    def wo_copy(v, q, slot):
        return pltpu.make_async_copy(
            wo_hbm.at[pl.ds(q * KQ, KQ), pl.ds(v * PV, PV)],
            worh_ref.at[slot], wo_sem.at[slot])

    def out_copy(v, c, slot):
        return pltpu.make_async_copy(
            outr_ref.at[slot],
            out_hbm.at[0, pl.ds(c * PT, PT), pl.ds(v * PV, PV)],
            out_sem.at[slot])

    wo_copy(0, 0, 0).start()
    wo_copy(0, 1, 1).start()
    pltpu.make_async_copy(wo_hbm.at[:, pl.ds(NFULL * PV, REM)], worem_ref,
                          rem_sem).start()

    def vloop(v, _):
        for q in range(4):
            slot = q % 2
            wo_copy(v, q, slot).wait()
            wobf_ref[pl.ds(q * KQ, KQ), :] = worh_ref[slot].astype(jnp.bfloat16)
            if q >= 2:
                @pl.when(v + 1 < NFULL)
                def _():
                    wo_copy(v + 1, q - 2, slot).start()
            else:
                wo_copy(v, q + 2, slot).start()

        wob = wobf_ref[...]
        bt = bo_ref[:, pl.ds(v * PV, PV)]

        def cloop(c, _):
            slot = lax.rem(c, ORING)
            i = v * NSL + c

            @pl.when(i >= ORING)
            def _():
                out_copy(0, 0, slot).wait()

            res = jnp.dot(y_ref[pl.ds(c * PT, PT), :], wob,
                          preferred_element_type=jnp.float32)
            outr_ref[slot] = res + bt
            out_copy(v, c, slot).start()
            return 0

        lax.fori_loop(0, NSL, cloop, 0, unroll=False)
        return 0

    lax.fori_loop(0, NFULL, vloop, 0, unroll=False)
    for k in range(ORING):
        out_copy(0, 0, k).wait()

    pltpu.make_async_copy(wo_hbm.at[:, pl.ds(NFULL * PV, REM)], worem_ref,
                          rem_sem).wait()
    btr = bo_ref[:, pl.ds(NFULL * PV, REM)]

    def rem_copy(c, slot):
        return pltpu.make_async_copy(
            outrem_ref.at[slot],
            out_hbm.at[0, pl.ds(c * 128, 128), pl.ds(NFULL * PV, REM)],
            out_sem.at[slot])

    def rloop(c, _):
        slot = lax.rem(c, 2)

        @pl.when(c >= 2)
        def _():
            rem_copy(0, slot).wait()

        res = jnp.dot(y_ref[pl.ds(c * 128, 128), :],
                      worem_ref[...].astype(jnp.bfloat16),
                      preferred_element_type=jnp.float32)
        outrem_ref[slot] = res + btr
        rem_copy(c, slot).start()
        return 0

    lax.fori_loop(0, N_TOK // 128, rloop, 0, unroll=False)
    rem_copy(0, 0).wait()
    rem_copy(0, 1).wait()


def _proj(y, wo, bo2d, vocab):
    return pl.pallas_call(
        _proj_manual_body,
        in_specs=[
            pl.BlockSpec((N_TOK, DIM), lambda: (0, 0)),
            pl.BlockSpec(memory_space=pltpu.HBM),
            pl.BlockSpec((1, vocab), lambda: (0, 0)),
        ],
        out_specs=pl.BlockSpec(memory_space=pltpu.HBM),
        out_shape=jax.ShapeDtypeStruct((1, N_TOK, vocab), jnp.float32),
        scratch_shapes=[
            pltpu.VMEM((ORING, PT, PV), jnp.float32),
            pltpu.VMEM((2, DIM // 4, PV), jnp.float32),
            pltpu.VMEM((DIM, PV), jnp.bfloat16),
            pltpu.VMEM((DIM, REM), jnp.float32),
            pltpu.VMEM((2, 128, REM), jnp.float32),
            pltpu.SemaphoreType.DMA((ORING,)),
            pltpu.SemaphoreType.DMA((2,)),
            pltpu.SemaphoreType.DMA,
        ],
        compiler_params=pltpu.CompilerParams(
            vmem_limit_bytes=128 * 1024 * 1024),
    )(y, wo, bo2d)


def kernel(x, emb, Wg, W1, b1, W2, b2, Wo, bo):
    b, t = x.shape
    vocab = Wo.shape[1]
    idx = x.reshape(-1).astype(jnp.int32)
    h = _sc_gather(emb, idx)
    cmb, aux = _router(h, Wg)
    y = _moe(h, cmb, W1, b1, W2, b2)
    logits = _proj(y, Wo, bo.reshape(1, -1), vocab)
    return logits, aux.reshape(())


# speedup vs baseline: 1.2860x; 1.2860x over previous
import functools

import jax
import jax.numpy as jnp
from jax import lax
from jax.experimental import pallas as pl
from jax.experimental.pallas import tpu as pltpu
from jax.experimental.pallas import tpu_sc as plsc

N_TOK = 2048
DIM = 768
NEXP = 8
HID = 4 * DIM
TOK_TILE = 256
VOCAB_TILE = 2048
PROJ_TOK = 512
PROJ_VOC = 4096


def _sc_gather(table, idx):
    info = plsc.get_sparse_core_info()
    nw = info.num_cores * info.num_subcores
    n = idx.shape[0]
    d = table.shape[1]
    b_per_w = n // nw
    mesh = plsc.VectorSubcoreMesh(core_axis_name="c", subcore_axis_name="s")

    @functools.partial(
        pl.kernel,
        mesh=mesh,
        out_type=jax.ShapeDtypeStruct((n, d), jnp.float32),
        scratch_types=[
            pltpu.VMEM((b_per_w,), jnp.int32),
            pltpu.VMEM((b_per_w, d), jnp.float32),
            pltpu.SemaphoreType.DMA,
        ],
    )
    def k(table_hbm, idx_hbm, out_hbm, idx_v, rows_v, sem):
        wid = lax.axis_index("s") * info.num_cores + lax.axis_index("c")
        base = wid * b_per_w
        pltpu.sync_copy(idx_hbm.at[pl.ds(base, b_per_w)], idx_v)
        pltpu.async_copy(table_hbm.at[idx_v], rows_v, sem).wait()
        pltpu.sync_copy(rows_v, out_hbm.at[pl.ds(base, b_per_w)])

    return k(table, idx)


def _router_body(h_ref, wg_ref, cmb_ref, aux_ref):
    h = h_ref[...]
    wg = wg_ref[...]
    logits = jnp.dot(h, wg, preferred_element_type=jnp.float32)
    probs = jax.nn.softmax(logits, axis=-1)
    gate = jnp.max(probs, axis=-1, keepdims=True)
    ids = lax.broadcasted_iota(jnp.int32, (N_TOK, NEXP), 1)
    eidx = jnp.min(jnp.where(probs >= gate, ids, NEXP), axis=-1, keepdims=True)
    oh = (ids == eidx).astype(jnp.float32)
    f = jnp.mean(oh, axis=0, keepdims=True)
    p_mean = jnp.mean(probs, axis=0, keepdims=True)
    aux_ref[...] = NEXP * jnp.sum(f * p_mean, axis=1, keepdims=True)
    cmb_ref[...] = oh * gate


def _router(h, wg):
    return pl.pallas_call(
        _router_body,
        out_shape=(
            jax.ShapeDtypeStruct((N_TOK, NEXP), jnp.float32),
            jax.ShapeDtypeStruct((1, 1), jnp.float32),
        ),
    )(h, wg)


def _moe_body(h_ref, cmb_ref, w1_ref, b1_ref, w2_ref, b2_ref, y_ref, acc_ref):
    e = pl.program_id(0)
    t = pl.program_id(1)
    hb = h_ref[...].astype(jnp.bfloat16)
    h1 = jnp.dot(hb, w1_ref[0].astype(jnp.bfloat16),
                 preferred_element_type=jnp.float32) + b1_ref[0]
    a = jax.nn.gelu(h1).astype(jnp.bfloat16)
    eo = jnp.dot(a, w2_ref[0].astype(jnp.bfloat16),
                 preferred_element_type=jnp.float32) + b2_ref[0]
    lane = lax.broadcasted_iota(jnp.int32, (TOK_TILE, NEXP), 1)
    cmb_e = jnp.sum(jnp.where(lane == e, cmb_ref[...], 0.0), axis=1,
                    keepdims=True)
    contrib = eo * cmb_e
    sl = pl.ds(t * TOK_TILE, TOK_TILE)

    @pl.when(e == 0)
    def _():
        acc_ref[sl, :] = contrib

    @pl.when(e > 0)
    def _():
        acc_ref[sl, :] = acc_ref[sl, :] + contrib

    @pl.when(e == NEXP - 1)
    def _():
        y_ref[...] = acc_ref[sl, :].astype(jnp.bfloat16)


def _moe(h, cmb, w1, b1, w2, b2):
    nt = N_TOK // TOK_TILE
    return pl.pallas_call(
        _moe_body,
        grid=(NEXP, nt),
        in_specs=[
            pl.BlockSpec((TOK_TILE, DIM), lambda e, t: (t, 0)),
            pl.BlockSpec((TOK_TILE, NEXP), lambda e, t: (t, 0)),
            pl.BlockSpec((1, DIM, HID), lambda e, t: (e, 0, 0)),
            pl.BlockSpec((1, 1, HID), lambda e, t: (e, 0, 0)),
            pl.BlockSpec((1, HID, DIM), lambda e, t: (e, 0, 0)),
            pl.BlockSpec((1, 1, DIM), lambda e, t: (e, 0, 0)),
        ],
        out_specs=pl.BlockSpec((TOK_TILE, DIM), lambda e, t: (t, 0)),
        out_shape=jax.ShapeDtypeStruct((N_TOK, DIM), jnp.bfloat16),
        scratch_shapes=[pltpu.VMEM((N_TOK, DIM), jnp.float32)],
    )(h, cmb, w1.reshape(NEXP, DIM, HID), b1.reshape(NEXP, 1, HID),
      w2.reshape(NEXP, HID, DIM), b2.reshape(NEXP, 1, DIM))


PV = 8192
PT = 256
NSL = N_TOK // PT
NFULL = 12
REM = 1696
ORING = 2


KQ = DIM // 8


def _proj_manual_body(y_ref, wo_hbm, bo_ref, out_hbm,
                      outr_ref, worh_ref, wobf_ref, worem_ref, outrem_ref,
                      out_sem, wo_sem, rem_sem):

    def wo_copy(v, q, slot):
        return pltpu.make_async_copy(
            wo_hbm.at[pl.ds(q * KQ, KQ), pl.ds(v * PV, PV)],
            worh_ref.at[slot], wo_sem.at[slot])

    def out_copy(v, c, slot):
        return pltpu.make_async_copy(
            outr_ref.at[slot],
            out_hbm.at[0, pl.ds(c * PT, PT), pl.ds(v * PV, PV)],
            out_sem.at[slot])

    wo_copy(0, 0, 0).start()
    wo_copy(0, 1, 1).start()
    pltpu.make_async_copy(wo_hbm.at[:, pl.ds(NFULL * PV, REM)], worem_ref,
                          rem_sem).start()

    def vloop(v, _):
        for q in range(8):
            slot = q % 2
            wo_copy(v, q, slot).wait()
            wobf_ref[pl.ds(q * KQ, KQ), :] = worh_ref[slot].astype(jnp.bfloat16)
            if q >= 6:
                @pl.when(v + 1 < NFULL)
                def _():
                    wo_copy(v + 1, q - 6, slot).start()
            else:
                wo_copy(v, q + 2, slot).start()

        wob = wobf_ref[...]
        bt = bo_ref[:, pl.ds(v * PV, PV)]

        def cloop(c, _):
            slot = lax.rem(c, ORING)
            i = v * NSL + c

            @pl.when(i >= ORING)
            def _():
                out_copy(0, 0, slot).wait()

            res = jnp.dot(y_ref[pl.ds(c * PT, PT), :], wob,
                          preferred_element_type=jnp.float32)
            outr_ref[slot] = res + bt
            out_copy(v, c, slot).start()
            return 0

        lax.fori_loop(0, NSL, cloop, 0, unroll=False)
        return 0

    lax.fori_loop(0, NFULL, vloop, 0, unroll=False)
    for k in range(ORING):
        out_copy(0, 0, k).wait()

    pltpu.make_async_copy(wo_hbm.at[:, pl.ds(NFULL * PV, REM)], worem_ref,
                          rem_sem).wait()
    btr = bo_ref[:, pl.ds(NFULL * PV, REM)]

    def rem_copy(c, slot):
        return pltpu.make_async_copy(
            outrem_ref.at[slot],
            out_hbm.at[0, pl.ds(c * 64, 64), pl.ds(NFULL * PV, REM)],
            out_sem.at[slot])

    def rloop(c, _):
        slot = lax.rem(c, 2)

        @pl.when(c >= 2)
        def _():
            rem_copy(0, slot).wait()

        res = jnp.dot(y_ref[pl.ds(c * 64, 64), :],
                      worem_ref[...].astype(jnp.bfloat16),
                      preferred_element_type=jnp.float32)
        outrem_ref[slot] = res + btr
        rem_copy(c, slot).start()
        return 0

    lax.fori_loop(0, N_TOK // 64, rloop, 0, unroll=False)
    rem_copy(0, 0).wait()
    rem_copy(0, 1).wait()


def _proj(y, wo, bo2d, vocab):
    return pl.pallas_call(
        _proj_manual_body,
        in_specs=[
            pl.BlockSpec((N_TOK, DIM), lambda: (0, 0)),
            pl.BlockSpec(memory_space=pltpu.HBM),
            pl.BlockSpec((1, vocab), lambda: (0, 0)),
        ],
        out_specs=pl.BlockSpec(memory_space=pltpu.HBM),
        out_shape=jax.ShapeDtypeStruct((1, N_TOK, vocab), jnp.float32),
        scratch_shapes=[
            pltpu.VMEM((ORING, PT, PV), jnp.float32),
            pltpu.VMEM((2, DIM // 8, PV), jnp.float32),
            pltpu.VMEM((DIM, PV), jnp.bfloat16),
            pltpu.VMEM((DIM, REM), jnp.float32),
            pltpu.VMEM((2, 64, REM), jnp.float32),
            pltpu.SemaphoreType.DMA((ORING,)),
            pltpu.SemaphoreType.DMA((2,)),
            pltpu.SemaphoreType.DMA,
        ],
        compiler_params=pltpu.CompilerParams(
            vmem_limit_bytes=128 * 1024 * 1024),
    )(y, wo, bo2d)


def kernel(x, emb, Wg, W1, b1, W2, b2, Wo, bo):
    b, t = x.shape
    vocab = Wo.shape[1]
    idx = x.reshape(-1).astype(jnp.int32)
    h = _sc_gather(emb, idx)
    cmb, aux = _router(h, Wg)
    y = _moe(h, cmb, W1, b1, W2, b2)
    logits = _proj(y, Wo, bo.reshape(1, -1), vocab)
    return logits, aux.reshape(())
